# baseline (speedup 1.0000x reference)
import functools

import jax
import jax.numpy as jnp
from jax import lax
from jax.experimental import pallas as pl
from jax.experimental.pallas import tpu as pltpu
from jax.experimental.pallas import tpu_sc as plsc

B, S, F = 4, 4096, 2048
CTX = 2048
ROW0 = S - CTX
NINF = float("-inf")

RT = 1536
RS_SC = CTX - RT

NC, NS, L = 2, 16, 16
NW = NC * NS
SEGS = NW // B
RPW = RS_SC // SEGS
R = 8
NCHUNK = RPW // R
NJ = F // L

_MESH = plsc.VectorSubcoreMesh(
    core_axis_name="c", subcore_axis_name="s", num_cores=NC, num_subcores=NS
)


@functools.partial(
    pl.kernel,
    out_type=tuple(
        jax.ShapeDtypeStruct((B, SEGS, F), jnp.float32) for _ in range(4)
    ),
    mesh=_MESH,
    scratch_types=[
        pltpu.VMEM((2, R, F), jnp.float32),
        pltpu.VMEM((2, R, F), jnp.float32),
        pltpu.VMEM((F,), jnp.float32),
        pltpu.VMEM((F,), jnp.float32),
        pltpu.VMEM((F,), jnp.float32),
        pltpu.VMEM((F,), jnp.float32),
        pltpu.SemaphoreType.DMA,
        pltpu.SemaphoreType.DMA,
        pltpu.SemaphoreType.DMA,
        pltpu.SemaphoreType.DMA,
    ],
)
def _sc_part(data_hbm, mask_hbm, pdm_hbm, pm_hbm, pd_hbm, pmx_hbm,
             dbuf, mbuf, acc_dm, acc_m, acc_d, acc_mx,
             sd0, sd1, sm0, sm1):
    wid = lax.axis_index("s") * NC + lax.axis_index("c")
    b = wid // SEGS
    seg = wid % SEGS
    row_base = ROW0 + RT + seg * RPW
    sems_d = (sd0, sd1)
    sems_m = (sm0, sm1)

    def src_d(ck):
        return data_hbm.at[b, pl.ds(row_base + ck * R, R), pl.ds(0, F)]

    def src_m(ck):
        return mask_hbm.at[b, pl.ds(row_base + ck * R, R), pl.ds(0, F)]

    def start(ck, buf):
        pltpu.async_copy(src_d(ck), dbuf.at[buf], sems_d[buf])
        pltpu.async_copy(src_m(ck), mbuf.at[buf], sems_m[buf])

    def wait(buf):
        pltpu.make_async_copy(src_d(0), dbuf.at[buf], sems_d[buf]).wait()
        pltpu.make_async_copy(src_m(0), mbuf.at[buf], sems_m[buf]).wait()

    zeros = jnp.zeros((L,), jnp.float32)
    ninf = jnp.full((L,), NINF, jnp.float32)

    @plsc.parallel_loop(0, NJ, unroll=4)
    def _(j):
        sl = pl.ds(j * L, L)
        acc_dm[sl] = zeros
        acc_m[sl] = zeros
        acc_d[sl] = zeros
        acc_mx[sl] = ninf

    def compute(buf):
        @plsc.parallel_loop(0, NJ, unroll=2)
        def _(j):
            sl = pl.ds(j * L, L)
            a_dm = acc_dm[sl]
            a_m = acc_m[sl]
            a_d = acc_d[sl]
            a_mx = acc_mx[sl]
            for r in range(R):
                d = dbuf[buf, r, sl]
                m = mbuf[buf, r, sl]
                a_dm = a_dm + d * m
                a_m = a_m + m
                a_d = a_d + d
                a_mx = jnp.maximum(a_mx, m)
            acc_dm[sl] = a_dm
            acc_m[sl] = a_m
            acc_d[sl] = a_d
            acc_mx[sl] = a_mx

    start(0, 0)
    start(1, 1)

    def pair_body(cp, carry):
        g0 = 2 * cp
        wait(0)
        compute(0)
        start(g0 + 2, 0)
        wait(1)
        compute(1)
        start(g0 + 3, 1)
        return carry

    lax.fori_loop(0, NCHUNK // 2 - 1, pair_body, 0)
    wait(0)
    compute(0)
    wait(1)
    compute(1)

    pltpu.sync_copy(acc_dm, pdm_hbm.at[b, seg, pl.ds(0, F)])
    pltpu.sync_copy(acc_m, pm_hbm.at[b, seg, pl.ds(0, F)])
    pltpu.sync_copy(acc_d, pd_hbm.at[b, seg, pl.ds(0, F)])
    pltpu.sync_copy(acc_mx, pmx_hbm.at[b, seg, pl.ds(0, F)])


BR = 256
NRT = RT // BR


def _tc_body(d_ref, m_ref, pdm_ref, pm_ref, pd_ref, pmx_ref,
             adm, am, ad, amx):
    r = pl.program_id(1)

    @pl.when(r == 0)
    def _init():
        adm[...] = jnp.zeros_like(adm)
        am[...] = jnp.zeros_like(am)
        ad[...] = jnp.zeros_like(ad)
        amx[...] = jnp.full_like(amx, NINF)

    a_dm = adm[...]
    a_m = am[...]
    a_d = ad[...]
    a_mx = amx[...]
    for i in range(BR // 8):
        sl = pl.ds(i * 8, 8)
        d = d_ref[0, sl]
        m = m_ref[0, sl]
        a_dm += d * m
        a_m += m
        a_d += d
        a_mx = jnp.maximum(a_mx, m)
    adm[...] = a_dm
    am[...] = a_m
    ad[...] = a_d
    amx[...] = a_mx

    @pl.when(r == NRT - 1)
    def _fin():
        pdm_ref[0] = adm[...]
        pm_ref[0] = am[...]
        pd_ref[0] = ad[...]
        pmx_ref[0] = amx[...]


_tc_part = pl.pallas_call(
    _tc_body,
    grid=(B, NRT),
    in_specs=[
        pl.BlockSpec((1, BR, F), lambda b, r: (b, ROW0 // BR + r, 0)),
        pl.BlockSpec((1, BR, F), lambda b, r: (b, ROW0 // BR + r, 0)),
    ],
    out_specs=[
        pl.BlockSpec((1, 8, F), lambda b, r: (b, 0, 0)),
        pl.BlockSpec((1, 8, F), lambda b, r: (b, 0, 0)),
        pl.BlockSpec((1, 8, F), lambda b, r: (b, 0, 0)),
        pl.BlockSpec((1, 8, F), lambda b, r: (b, 0, 0)),
    ],
    out_shape=[
        jax.ShapeDtypeStruct((B, 8, F), jnp.float32),
        jax.ShapeDtypeStruct((B, 8, F), jnp.float32),
        jax.ShapeDtypeStruct((B, 8, F), jnp.float32),
        jax.ShapeDtypeStruct((B, 8, F), jnp.float32),
    ],
    scratch_shapes=[
        pltpu.VMEM((8, F), jnp.float32),
        pltpu.VMEM((8, F), jnp.float32),
        pltpu.VMEM((8, F), jnp.float32),
        pltpu.VMEM((8, F), jnp.float32),
    ],
)


def _combine_body(tdm, tm, td, tmx, sdm, sm, sd, smx, pooled_ref, mmax_ref):
    dm = jnp.sum(tdm[0], axis=0) + jnp.sum(sdm[0], axis=0)
    msum = jnp.sum(tm[0], axis=0) + jnp.sum(sm[0], axis=0)
    dsum = jnp.sum(td[0], axis=0) + jnp.sum(sd[0], axis=0)
    mx = jnp.maximum(jnp.max(tmx[0], axis=0), jnp.max(smx[0], axis=0))
    pooled_ref[0, 0] = jnp.where(
        msum == 0.0,
        dsum * jnp.float32(1.0 / CTX),
        dm / (msum + jnp.float32(1e-8)),
    )
    mmax_ref[0, 0] = mx


_combine = pl.pallas_call(
    _combine_body,
    grid=(B,),
    in_specs=[pl.BlockSpec((1, 8, F), lambda b: (b, 0, 0))] * 4
    + [pl.BlockSpec((1, SEGS, F), lambda b: (b, 0, 0))] * 4,
    out_specs=[
        pl.BlockSpec((1, 1, F), lambda b: (b, 0, 0)),
        pl.BlockSpec((1, 1, F), lambda b: (b, 0, 0)),
    ],
    out_shape=[
        jax.ShapeDtypeStruct((B, 1, F), jnp.float32),
        jax.ShapeDtypeStruct((B, 1, F), jnp.float32),
    ],
)


def kernel(data, mask):
    sdm, sm, sd, smx = _sc_part(data, mask)
    tdm, tm, td, tmx = _tc_part(data, mask)
    pooled, mmax = _combine(tdm, tm, td, tmx, sdm, sm, sd, smx)
    return (pooled, mmax)


# speedup vs baseline: 1.1219x; 1.1219x over previous
import functools

import jax
import jax.numpy as jnp
from jax import lax
from jax.experimental import pallas as pl
from jax.experimental.pallas import tpu as pltpu
from jax.experimental.pallas import tpu_sc as plsc

B, S, F = 4, 4096, 2048
CTX = 2048
ROW0 = S - CTX
NINF = float("-inf")

RT = 1280
RS_SC = CTX - RT

NC, NS, L = 2, 16, 16
NW = NC * NS
SEGS = NW // B
RPW = RS_SC // SEGS
R = 8
NCHUNK = RPW // R
NJ = F // L

_MESH = plsc.VectorSubcoreMesh(
    core_axis_name="c", subcore_axis_name="s", num_cores=NC, num_subcores=NS
)


@functools.partial(
    pl.kernel,
    out_type=tuple(
        jax.ShapeDtypeStruct((B, SEGS, F), jnp.float32) for _ in range(4)
    ),
    mesh=_MESH,
    scratch_types=[
        pltpu.VMEM((2, R, F), jnp.float32),
        pltpu.VMEM((2, R, F), jnp.float32),
        pltpu.VMEM((F,), jnp.float32),
        pltpu.VMEM((F,), jnp.float32),
        pltpu.VMEM((F,), jnp.float32),
        pltpu.VMEM((F,), jnp.float32),
        pltpu.SemaphoreType.DMA,
        pltpu.SemaphoreType.DMA,
        pltpu.SemaphoreType.DMA,
        pltpu.SemaphoreType.DMA,
    ],
)
def _sc_part(data_hbm, mask_hbm, pdm_hbm, pm_hbm, pd_hbm, pmx_hbm,
             dbuf, mbuf, acc_dm, acc_m, acc_d, acc_mx,
             sd0, sd1, sm0, sm1):
    wid = lax.axis_index("s") * NC + lax.axis_index("c")
    b = wid // SEGS
    seg = wid % SEGS
    row_base = ROW0 + RT + seg * RPW
    sems_d = (sd0, sd1)
    sems_m = (sm0, sm1)

    def src_d(ck):
        return data_hbm.at[b, pl.ds(row_base + ck * R, R), pl.ds(0, F)]

    def src_m(ck):
        return mask_hbm.at[b, pl.ds(row_base + ck * R, R), pl.ds(0, F)]

    def start(ck, buf):
        pltpu.async_copy(src_d(ck), dbuf.at[buf], sems_d[buf])
        pltpu.async_copy(src_m(ck), mbuf.at[buf], sems_m[buf])

    def wait(buf):
        pltpu.make_async_copy(src_d(0), dbuf.at[buf], sems_d[buf]).wait()
        pltpu.make_async_copy(src_m(0), mbuf.at[buf], sems_m[buf]).wait()

    zeros = jnp.zeros((L,), jnp.float32)
    ninf = jnp.full((L,), NINF, jnp.float32)

    @plsc.parallel_loop(0, NJ, unroll=4)
    def _(j):
        sl = pl.ds(j * L, L)
        acc_dm[sl] = zeros
        acc_m[sl] = zeros
        acc_d[sl] = zeros
        acc_mx[sl] = ninf

    def compute(buf):
        @plsc.parallel_loop(0, NJ, unroll=2)
        def _(j):
            sl = pl.ds(j * L, L)
            a_dm = acc_dm[sl]
            a_m = acc_m[sl]
            a_d = acc_d[sl]
            a_mx = acc_mx[sl]
            for r in range(R):
                d = dbuf[buf, r, sl]
                m = mbuf[buf, r, sl]
                a_dm = a_dm + d * m
                a_m = a_m + m
                a_d = a_d + d
                a_mx = jnp.maximum(a_mx, m)
            acc_dm[sl] = a_dm
            acc_m[sl] = a_m
            acc_d[sl] = a_d
            acc_mx[sl] = a_mx

    start(0, 0)
    start(1, 1)

    def pair_body(cp, carry):
        g0 = 2 * cp
        wait(0)
        compute(0)
        start(g0 + 2, 0)
        wait(1)
        compute(1)
        start(g0 + 3, 1)
        return carry

    lax.fori_loop(0, NCHUNK // 2 - 1, pair_body, 0)
    wait(0)
    compute(0)
    wait(1)
    compute(1)

    pltpu.sync_copy(acc_dm, pdm_hbm.at[b, seg, pl.ds(0, F)])
    pltpu.sync_copy(acc_m, pm_hbm.at[b, seg, pl.ds(0, F)])
    pltpu.sync_copy(acc_d, pd_hbm.at[b, seg, pl.ds(0, F)])
    pltpu.sync_copy(acc_mx, pmx_hbm.at[b, seg, pl.ds(0, F)])


BR = 256
NRT = RT // BR


def _tc_body(d_ref, m_ref, pdm_ref, pm_ref, pd_ref, pmx_ref,
             adm, am, ad, amx):
    r = pl.program_id(1)

    @pl.when(r == 0)
    def _init():
        adm[...] = jnp.zeros_like(adm)
        am[...] = jnp.zeros_like(am)
        ad[...] = jnp.zeros_like(ad)
        amx[...] = jnp.full_like(amx, NINF)

    a_dm = adm[...]
    a_m = am[...]
    a_d = ad[...]
    a_mx = amx[...]
    for i in range(BR // 8):
        sl = pl.ds(i * 8, 8)
        d = d_ref[0, sl]
        m = m_ref[0, sl]
        a_dm += d * m
        a_m += m
        a_d += d
        a_mx = jnp.maximum(a_mx, m)
    adm[...] = a_dm
    am[...] = a_m
    ad[...] = a_d
    amx[...] = a_mx

    @pl.when(r == NRT - 1)
    def _fin():
        pdm_ref[0] = adm[...]
        pm_ref[0] = am[...]
        pd_ref[0] = ad[...]
        pmx_ref[0] = amx[...]


_tc_part = pl.pallas_call(
    _tc_body,
    grid=(B, NRT),
    in_specs=[
        pl.BlockSpec((1, BR, F), lambda b, r: (b, ROW0 // BR + r, 0)),
        pl.BlockSpec((1, BR, F), lambda b, r: (b, ROW0 // BR + r, 0)),
    ],
    out_specs=[
        pl.BlockSpec((1, 8, F), lambda b, r: (b, 0, 0)),
        pl.BlockSpec((1, 8, F), lambda b, r: (b, 0, 0)),
        pl.BlockSpec((1, 8, F), lambda b, r: (b, 0, 0)),
        pl.BlockSpec((1, 8, F), lambda b, r: (b, 0, 0)),
    ],
    out_shape=[
        jax.ShapeDtypeStruct((B, 8, F), jnp.float32),
        jax.ShapeDtypeStruct((B, 8, F), jnp.float32),
        jax.ShapeDtypeStruct((B, 8, F), jnp.float32),
        jax.ShapeDtypeStruct((B, 8, F), jnp.float32),
    ],
    scratch_shapes=[
        pltpu.VMEM((8, F), jnp.float32),
        pltpu.VMEM((8, F), jnp.float32),
        pltpu.VMEM((8, F), jnp.float32),
        pltpu.VMEM((8, F), jnp.float32),
    ],
)


def _combine_body(tdm, tm, td, tmx, sdm, sm, sd, smx, pooled_ref, mmax_ref):
    dm = jnp.sum(tdm[0], axis=0) + jnp.sum(sdm[0], axis=0)
    msum = jnp.sum(tm[0], axis=0) + jnp.sum(sm[0], axis=0)
    dsum = jnp.sum(td[0], axis=0) + jnp.sum(sd[0], axis=0)
    mx = jnp.maximum(jnp.max(tmx[0], axis=0), jnp.max(smx[0], axis=0))
    pooled_ref[0, 0] = jnp.where(
        msum == 0.0,
        dsum * jnp.float32(1.0 / CTX),
        dm / (msum + jnp.float32(1e-8)),
    )
    mmax_ref[0, 0] = mx


_combine = pl.pallas_call(
    _combine_body,
    grid=(B,),
    in_specs=[pl.BlockSpec((1, 8, F), lambda b: (b, 0, 0))] * 4
    + [pl.BlockSpec((1, SEGS, F), lambda b: (b, 0, 0))] * 4,
    out_specs=[
        pl.BlockSpec((1, 1, F), lambda b: (b, 0, 0)),
        pl.BlockSpec((1, 1, F), lambda b: (b, 0, 0)),
    ],
    out_shape=[
        jax.ShapeDtypeStruct((B, 1, F), jnp.float32),
        jax.ShapeDtypeStruct((B, 1, F), jnp.float32),
    ],
)


def kernel(data, mask):
    sdm, sm, sd, smx = _sc_part(data, mask)
    tdm, tm, td, tmx = _tc_part(data, mask)
    pooled, mmax = _combine(tdm, tm, td, tmx, sdm, sm, sd, smx)
    return (pooled, mmax)
